# baseline (speedup 1.0000x reference)
import jax
import jax.numpy as jnp
from jax.experimental import pallas as pl
from jax.experimental.pallas import tpu as pltpu


def _plane_kernel(na_ref, col_ref, row_ref, out_ref):
    b = pl.program_id(0)
    na = na_ref[b]
    col = col_ref[0, 0]
    row = row_ref[0, 0]
    n = col.shape[0]
    jio = jax.lax.broadcasted_iota(jnp.int32, (n, n), 0)
    kio = jax.lax.broadcasted_iota(jnp.int32, (n, n), 1)
    mask = (jio < na) & (kio < na)
    out_ref[0, 0] = jnp.where(mask, col - row, jnp.float32(0.0))


def kernel(coords, num_atoms):
    bsz, flat = coords.shape
    maxa = flat // 3
    xt = coords.reshape(bsz, maxa, 3).transpose(0, 2, 1)
    xcol = xt.reshape(bsz, 3, maxa, 1)
    xrow = xt.reshape(bsz, 3, 1, maxa)
    na = num_atoms.astype(jnp.int32)
    out = pl.pallas_call(
        _plane_kernel,
        grid_spec=pltpu.PrefetchScalarGridSpec(
            num_scalar_prefetch=1,
            grid=(bsz, 3),
            in_specs=[
                pl.BlockSpec((1, 1, maxa, 1), lambda b, c, na_ref: (b, c, 0, 0)),
                pl.BlockSpec((1, 1, 1, maxa), lambda b, c, na_ref: (b, c, 0, 0)),
            ],
            out_specs=pl.BlockSpec((1, 1, maxa, maxa),
                                   lambda b, c, na_ref: (b, c, 0, 0)),
        ),
        out_shape=jax.ShapeDtypeStruct((bsz, 3, maxa, maxa), jnp.float32),
        compiler_params=pltpu.CompilerParams(
            dimension_semantics=("parallel", "parallel")),
    )(na, xcol, xrow)
    return out.transpose(0, 2, 3, 1)


# speedup vs baseline: 1.0702x; 1.0702x over previous
import jax
import jax.numpy as jnp
from jax.experimental import pallas as pl
from jax.experimental.pallas import tpu as pltpu

_NBUF = 4
_NC = 3


def _plane_kernel(na_ref, col_ref, row_ref, out_hbm, scratch, sems):
    b = pl.program_id(0)
    c = pl.program_id(1)
    nb = pl.num_programs(0)
    i = b * _NC + c
    total = nb * _NC
    slot = jax.lax.rem(i, _NBUF)

    @pl.when(i >= _NBUF)
    def _wait_prev():
        prev = i - _NBUF
        pltpu.make_async_copy(
            scratch.at[slot],
            out_hbm.at[prev // _NC, jax.lax.rem(prev, _NC)],
            sems.at[slot],
        ).wait()

    na = na_ref[b]
    col = col_ref[0, 0]
    row = row_ref[0, 0]
    n = col.shape[0]
    jio = jax.lax.broadcasted_iota(jnp.int32, (n, n), 0)
    kio = jax.lax.broadcasted_iota(jnp.int32, (n, n), 1)
    mask = (jio < na) & (kio < na)
    scratch[slot] = jnp.where(mask, col - row, jnp.float32(0.0))

    pltpu.make_async_copy(scratch.at[slot], out_hbm.at[b, c],
                          sems.at[slot]).start()

    @pl.when(i == total - 1)
    def _drain():
        for s in range(_NBUF):
            prev = total - _NBUF + s
            pltpu.make_async_copy(
                scratch.at[jax.lax.rem(jnp.int32(prev), _NBUF)],
                out_hbm.at[prev // _NC, prev % _NC],
                sems.at[jax.lax.rem(jnp.int32(prev), _NBUF)],
            ).wait()


def kernel(coords, num_atoms):
    bsz, flat = coords.shape
    maxa = flat // 3
    xt = coords.reshape(bsz, maxa, 3).transpose(0, 2, 1)
    xcol = xt.reshape(bsz, 3, maxa, 1)
    xrow = xt.reshape(bsz, 3, 1, maxa)
    na = num_atoms.astype(jnp.int32)
    out = pl.pallas_call(
        _plane_kernel,
        grid_spec=pltpu.PrefetchScalarGridSpec(
            num_scalar_prefetch=1,
            grid=(bsz, _NC),
            in_specs=[
                pl.BlockSpec((1, 1, maxa, 1), lambda b, c, na_ref: (b, c, 0, 0)),
                pl.BlockSpec((1, 1, 1, maxa), lambda b, c, na_ref: (b, c, 0, 0)),
            ],
            out_specs=pl.BlockSpec(memory_space=pl.ANY),
            scratch_shapes=[
                pltpu.VMEM((_NBUF, maxa, maxa), jnp.float32),
                pltpu.SemaphoreType.DMA((_NBUF,)),
            ],
        ),
        out_shape=jax.ShapeDtypeStruct((bsz, _NC, maxa, maxa), jnp.float32),
    )(na, xcol, xrow)
    return out.transpose(0, 2, 3, 1)


# speedup vs baseline: 1.4274x; 1.3338x over previous
import jax
import jax.numpy as jnp
from jax.experimental import pallas as pl
from jax.experimental.pallas import tpu as pltpu

_NC = 3
_NSET = 2


def _plane_kernel(na_ref, col_ref, row_ref, out_hbm, scratch, sems):
    b = pl.program_id(0)
    nb = pl.num_programs(0)
    sset = jax.lax.rem(b, _NSET)
    na = na_ref[b]

    n = col_ref.shape[2]
    jio = jax.lax.broadcasted_iota(jnp.int32, (n, n), 0)
    kio = jax.lax.broadcasted_iota(jnp.int32, (n, n), 1)
    mask = (jio < na) & (kio < na)

    @pl.when(b >= _NSET)
    def _wait_prev():
        for c in range(_NC):
            pltpu.make_async_copy(
                scratch.at[sset, c], out_hbm.at[b - _NSET, c],
                sems.at[sset, c]).wait()

    for c in range(_NC):
        col = col_ref[0, c]
        row = row_ref[0, c]
        scratch[sset, c] = jnp.where(mask, col - row, jnp.float32(0.0))
        pltpu.make_async_copy(scratch.at[sset, c], out_hbm.at[b, c],
                              sems.at[sset, c]).start()

    @pl.when(b == nb - 1)
    def _drain():
        for s in range(_NSET):
            prev = nb - _NSET + s
            for c in range(_NC):
                pltpu.make_async_copy(
                    scratch.at[jax.lax.rem(jnp.int32(prev), _NSET), c],
                    out_hbm.at[prev, c],
                    sems.at[jax.lax.rem(jnp.int32(prev), _NSET), c]).wait()


def kernel(coords, num_atoms):
    bsz, flat = coords.shape
    maxa = flat // 3
    xt = coords.reshape(bsz, maxa, 3).transpose(0, 2, 1)
    xcol = xt.reshape(bsz, 3, maxa, 1)
    xrow = xt.reshape(bsz, 3, 1, maxa)
    na = num_atoms.astype(jnp.int32)
    out = pl.pallas_call(
        _plane_kernel,
        grid_spec=pltpu.PrefetchScalarGridSpec(
            num_scalar_prefetch=1,
            grid=(bsz,),
            in_specs=[
                pl.BlockSpec((1, _NC, maxa, 1), lambda b, na_ref: (b, 0, 0, 0)),
                pl.BlockSpec((1, _NC, 1, maxa), lambda b, na_ref: (b, 0, 0, 0)),
            ],
            out_specs=pl.BlockSpec(memory_space=pl.ANY),
            scratch_shapes=[
                pltpu.VMEM((_NSET, _NC, maxa, maxa), jnp.float32),
                pltpu.SemaphoreType.DMA((_NSET, _NC)),
            ],
        ),
        out_shape=jax.ShapeDtypeStruct((bsz, _NC, maxa, maxa), jnp.float32),
    )(na, xcol, xrow)
    return out.transpose(0, 2, 3, 1)
